# baseline (speedup 1.0000x reference)
import functools

import jax
import jax.numpy as jnp
from jax import lax
from jax.experimental import pallas as pl
from jax.experimental.pallas import tpu as pltpu
from jax.experimental.pallas import tpu_sc as plsc

_D = 16
_BATCH = 16384
_FIELDS = 26
_B = _BATCH * _FIELDS
_NW = 32
_BB = _BATCH // _NW
_BPW = _BB * _FIELDS

_mesh = plsc.VectorSubcoreMesh(core_axis_name="c", subcore_axis_name="s")


@functools.partial(
    pl.kernel,
    mesh=_mesh,
    out_type=jax.ShapeDtypeStruct((_FIELDS, 2, _BATCH // 128, 1024),
                                  jnp.float32),
    compiler_params=pltpu.CompilerParams(use_tc_tiling_on_sc=False,
                                         needs_layout_passes=False),
    scratch_types=[
        pltpu.VMEM((_BPW,), jnp.int32),

        pltpu.VMEM((2, _BB, _D), jnp.float32),
        pltpu.VMEM((2, 4 * 2048), jnp.float32),
        pltpu.SemaphoreType.DMA,
        pltpu.SemaphoreType.DMA,
        pltpu.SemaphoreType.DMA,
        pltpu.SemaphoreType.DMA,
        pltpu.SemaphoreType.DMA,
    ],
)
def _embed_gather(eflat_hbm, table_hbm, out_hbm, idx_v, rows_v, planes_v,
                  isem, g0, g1, o0, o1):
    wid = lax.axis_index("s") * 2 + lax.axis_index("c")
    b0 = wid * _BB
    gsem = (g0, g1)
    osem = (o0, o1)

    ihs = [
        pltpu.async_copy(eflat_hbm.at[pl.ds(f * _BATCH + b0, _BB)],
                         idx_v.at[pl.ds(f * _BB, _BB)], isem)
        for f in range(_FIELDS)
    ]
    for h in ihs:
        h.wait()

    def start_gather(f):
        return pltpu.async_copy(
            table_hbm.at[idx_v.at[pl.ds(f * _BB, _BB)]],
            rows_v.at[f % 2], gsem[f % 2])

    gathers = [None, None]
    puts = [[], []]
    gathers[0] = start_gather(0)
    for f in range(_FIELDS):
        if f + 1 < _FIELDS:
            gathers[(f + 1) % 2] = start_gather(f + 1)
        gathers[f % 2].wait()

        buf = f % 2
        for h in puts[buf]:
            h.wait()
        puts[buf] = []
        pbuf = planes_v.at[buf]
        rbuf = rows_v.at[buf]

        def body(b, _):
            row = rbuf[b, :]
            pos = (lax.shift_right_logical(b, 7) * 2048
                   + lax.bitwise_and(b, 127))
            idx = lax.iota(jnp.int32, 16) * 128 + (
                jnp.full((16,), 0, jnp.int32) + pos)
            plsc.store_scatter(pbuf, [idx], row)
            return 0

        lax.fori_loop(0, _BB, body, 0)

        tc0 = wid * (_BB // 128)
        for q in range(_BB // 128):
            for tr in range(2):
                h = pltpu.async_copy(
                    pbuf.at[pl.ds(q * 2048 + tr * 1024, 1024)],
                    out_hbm.at[f, tr, tc0 + q],
                    osem[buf])
                puts[buf].append(h)
    for hs in puts:
        for h in hs:
            h.wait()


def kernel(e, table):
    idx = jnp.clip(e.T.reshape(_B), 0, jnp.int32(1000000 - 1))
    out6 = _embed_gather(idx, table)
    out5 = out6.reshape(_FIELDS, 2, _BATCH // 128, 8, 128)
    return out5.transpose(2, 4, 0, 1, 3).reshape(_BATCH, _FIELDS, _D)


# speedup vs baseline: 1.0022x; 1.0022x over previous
import functools

import jax
import jax.numpy as jnp
from jax import lax
from jax.experimental import pallas as pl
from jax.experimental.pallas import tpu as pltpu
from jax.experimental.pallas import tpu_sc as plsc

_D = 16
_BATCH = 16384
_FIELDS = 26
_B = _BATCH * _FIELDS
_NW = 32
_BB = _BATCH // _NW
_BPW = _BB * _FIELDS

_mesh = plsc.VectorSubcoreMesh(core_axis_name="c", subcore_axis_name="s")


@functools.partial(
    pl.kernel,
    mesh=_mesh,
    out_type=jax.ShapeDtypeStruct((_FIELDS, 2, _BATCH // 128, 1024),
                                  jnp.float32),
    compiler_params=pltpu.CompilerParams(use_tc_tiling_on_sc=False,
                                         needs_layout_passes=False),
    scratch_types=[
        pltpu.VMEM((_BPW,), jnp.int32),

        pltpu.VMEM((2, _BB, _D), jnp.float32),
        pltpu.VMEM((2, 4 * 2048), jnp.float32),
        pltpu.SemaphoreType.DMA,
        pltpu.SemaphoreType.DMA,
        pltpu.SemaphoreType.DMA,
        pltpu.SemaphoreType.DMA,
        pltpu.SemaphoreType.DMA,
    ],
)
def _embed_gather(eflat_hbm, table_hbm, out_hbm, idx_v, rows_v, planes_v,
                  isem, g0, g1, o0, o1):
    wid = lax.axis_index("s") * 2 + lax.axis_index("c")
    b0 = wid * _BB
    gsem = (g0, g1)
    osem = (o0, o1)

    ihs = [
        pltpu.async_copy(eflat_hbm.at[pl.ds(f * _BATCH + b0, _BB)],
                         idx_v.at[pl.ds(f * _BB, _BB)], isem)
        for f in range(_FIELDS)
    ]
    for h in ihs:
        h.wait()

    def start_gather(f):
        return pltpu.async_copy(
            table_hbm.at[idx_v.at[pl.ds(f * _BB, _BB)]],
            rows_v.at[f % 2], gsem[f % 2])

    gathers = [None, None]
    puts = [[], []]
    gathers[0] = start_gather(0)
    for f in range(_FIELDS):
        if f + 1 < _FIELDS:
            gathers[(f + 1) % 2] = start_gather(f + 1)
        gathers[f % 2].wait()

        buf = f % 2
        for h in puts[buf]:
            h.wait()
        puts[buf] = []
        pbuf = planes_v.at[buf]
        rbuf = rows_v.at[buf]

        def body(b, _):
            row = rbuf[b, :]
            pos = (lax.shift_right_logical(b, 7) * 2048
                   + lax.bitwise_and(b, 127))
            idx = lax.iota(jnp.int32, 16) * 128 + (
                jnp.full((16,), 0, jnp.int32) + pos)
            plsc.store_scatter(pbuf, [idx], row)
            return 0

        lax.fori_loop(0, _BB, body, 0)

        tc0 = wid * (_BB // 128)
        for q in range(_BB // 128):
            for tr in range(2):
                h = pltpu.async_copy(
                    pbuf.at[pl.ds(q * 2048 + tr * 1024, 1024)],
                    out_hbm.at[f, tr, tc0 + q],
                    osem[buf])
                puts[buf].append(h)
    for hs in puts:
        for h in hs:
            h.wait()


def kernel(e, table):
    idx = jnp.concatenate([e[:, f] for f in range(_FIELDS)])
    out6 = _embed_gather(idx, table)
    out5 = out6.reshape(_FIELDS, 2, _BATCH // 128, 8, 128)
    return out5.transpose(2, 4, 0, 1, 3).reshape(_BATCH, _FIELDS, _D)
